# Initial kernel scaffold; baseline (speedup 1.0000x reference)
import functools

import jax
import jax.numpy as jnp
from jax import lax
from jax.experimental import pallas as pl
from jax.experimental.pallas import tpu as pltpu
from jax.experimental.pallas import tpu_sc as plsc

N = 10000
E = 320000
H = 128
G = 128
DOUT = 10

NW = 32
LANES = 128
ROWS_PER_W = 79
E_PAD = NW * ROWS_PER_W * LANES
N_PAD = 10240
TILE_ROWS = N_PAD // 16
DUMP_ROW = N


def _hop_body(h_hbm, scat_hbm, dst_hbm, zeros_hbm, out_hbm,
              scat_v, dst_v, rows_v, acc_sh, sem):
    c = lax.axis_index("c")
    s = lax.axis_index("s")
    wid = s * 2 + c
    pltpu.sync_copy(zeros_hbm, acc_sh.at[pl.ds(s * TILE_ROWS, TILE_ROWS)])
    base = wid * ROWS_PER_W
    pltpu.sync_copy(scat_hbm.at[pl.ds(base, ROWS_PER_W)], scat_v)
    pltpu.sync_copy(dst_hbm.at[pl.ds(base, ROWS_PER_W)], dst_v)
    plsc.subcore_barrier()

    @pl.loop(0, ROWS_PER_W)
    def _(j):
        pltpu.async_copy(h_hbm.at[scat_v.at[j]], rows_v, sem).wait()
        pltpu.sync_copy(rows_v, acc_sh.at[dst_v.at[j]], add=True)

    plsc.subcore_barrier()
    pltpu.sync_copy(acc_sh.at[pl.ds(s * TILE_ROWS, TILE_ROWS)],
                    out_hbm.at[c, pl.ds(s * TILE_ROWS, TILE_ROWS)])


def _sc_hop(h, scat_rows, dst_rows, zeros_tile):
    mesh = plsc.VectorSubcoreMesh(core_axis_name="c", subcore_axis_name="s")
    return pl.kernel(
        _hop_body,
        out_type=jax.ShapeDtypeStruct((2, N_PAD, H), jnp.float32),
        mesh=mesh,
        scratch_types=[
            pltpu.VMEM((ROWS_PER_W, LANES), jnp.int32),
            pltpu.VMEM((ROWS_PER_W, LANES), jnp.int32),
            pltpu.VMEM((LANES, H), jnp.float32),
            pltpu.VMEM_SHARED((N_PAD, H), jnp.float32),
            pltpu.SemaphoreType.DMA,
        ],
    )(h, scat_rows, dst_rows, zeros_tile)


def _lin0_kernel(x_ref, w_ref, b_ref, o_ref):
    o_ref[...] = lax.dot_general(
        x_ref[...], w_ref[...], (((1,), (1,)), ((), ())),
        preferred_element_type=jnp.float32) + b_ref[...]


def _lin0(x, w0, b0):
    blk = 1000
    return pl.pallas_call(
        _lin0_kernel,
        grid=(N // blk,),
        in_specs=[
            pl.BlockSpec((blk, H), lambda i: (i, 0)),
            pl.BlockSpec((H, H), lambda i: (0, 0)),
            pl.BlockSpec((1, H), lambda i: (0, 0)),
        ],
        out_specs=pl.BlockSpec((blk, H), lambda i: (i, 0)),
        out_shape=jax.ShapeDtypeStruct((N, H), jnp.float32),
    )(x, w0, b0.reshape(1, H))


def _combine_kernel(a_ref, b_ref, o_ref):
    o_ref[...] = a_ref[...] + b_ref[...]


def _combine(p):
    blk = 1024
    return pl.pallas_call(
        _combine_kernel,
        grid=(N_PAD // blk,),
        in_specs=[
            pl.BlockSpec((blk, H), lambda i: (i, 0)),
            pl.BlockSpec((blk, H), lambda i: (i, 0)),
        ],
        out_specs=pl.BlockSpec((blk, H), lambda i: (i, 0)),
        out_shape=jax.ShapeDtypeStruct((N_PAD, H), jnp.float32),
    )(p[0], p[1])


def _head_kernel(bv_ref, h0_ref, h1_ref, q0_ref, q1_ref, eps_ref, wpt_ref,
                 bp_ref, o_ref, acc_ref, cnt_ref, *, nblocks):
    i = pl.program_id(0)

    @pl.when(i == 0)
    def _():
        acc_ref[...] = jnp.zeros_like(acc_ref)
        cnt_ref[...] = jnp.zeros_like(cnt_ref)

    bv = bv_ref[0]
    gids = lax.broadcasted_iota(jnp.int32, (G, bv.shape[1]), 0)
    s_mat = jnp.where(bv == gids, 1.0, 0.0)
    dot = functools.partial(lax.dot_general,
                            dimension_numbers=(((1,), (0,)), ((), ())),
                            preferred_element_type=jnp.float32)
    h2blk = q0_ref[...] + q1_ref[...]
    acc_ref[:, 0:H] += dot(s_mat, h0_ref[...])
    acc_ref[:, H:2 * H] += dot(s_mat, h1_ref[...])
    acc_ref[:, 2 * H:3 * H] += dot(s_mat, h2blk)
    cnt_ref[...] += jnp.broadcast_to(
        jnp.sum(s_mat, axis=1, keepdims=True), cnt_ref.shape)

    @pl.when(i == nblocks - 1)
    def _():
        c = jnp.maximum(cnt_ref[...], 1.0)
        scale = 1.0 + eps_ref[...]
        pooled = jnp.concatenate(
            [acc_ref[:, 0:H] * scale / c,
             acc_ref[:, H:2 * H] / c,
             acc_ref[:, 2 * H:3 * H] / c], axis=1)
        o_ref[...] = lax.dot_general(
            pooled, wpt_ref[...], (((1,), (0,)), ((), ())),
            preferred_element_type=jnp.float32) + bp_ref[...]


def _head(bv3, h0, h1, q0, q1, eps_b, wpt, bp_pad):
    blk = 1000
    nb = N // blk
    return pl.pallas_call(
        functools.partial(_head_kernel, nblocks=nb),
        grid=(nb,),
        in_specs=[
            pl.BlockSpec((1, 1, blk), lambda i: (i, 0, 0)),
            pl.BlockSpec((blk, H), lambda i: (i, 0)),
            pl.BlockSpec((blk, H), lambda i: (i, 0)),
            pl.BlockSpec((blk, H), lambda i: (i, 0)),
            pl.BlockSpec((blk, H), lambda i: (i, 0)),
            pl.BlockSpec((1, H), lambda i: (0, 0)),
            pl.BlockSpec((3 * H, H), lambda i: (0, 0)),
            pl.BlockSpec((1, H), lambda i: (0, 0)),
        ],
        out_specs=pl.BlockSpec((G, H), lambda i: (0, 0)),
        out_shape=jax.ShapeDtypeStruct((G, H), jnp.float32),
        scratch_shapes=[
            pltpu.VMEM((G, 3 * H), jnp.float32),
            pltpu.VMEM((G, H), jnp.float32),
        ],
    )(bv3, h0, h1, q0, q1, eps_b, wpt, bp_pad)


def _pad_edges(scat, dst):
    pad = E_PAD - E
    scat_p = jnp.concatenate(
        [scat, jnp.zeros((pad,), jnp.int32)]).reshape(E_PAD // LANES, LANES)
    dst_p = jnp.concatenate(
        [dst, jnp.full((pad,), DUMP_ROW, jnp.int32)]).reshape(
            E_PAD // LANES, LANES)
    return scat_p, dst_p


def kernel(x, agg_scatter_0, agg_node_index_0, agg_scatter_1,
           agg_node_index_1, batch_vec, eps, W0, b0, Wp, bp):
    h0 = _lin0(x, W0, b0)

    zeros_tile = jnp.zeros((TILE_ROWS, H), jnp.float32)
    s0, d0 = _pad_edges(agg_scatter_0, agg_node_index_0)
    s1, d1 = _pad_edges(agg_scatter_1, agg_node_index_1)

    p = _sc_hop(h0, s0, d0, zeros_tile)
    h1 = _combine(p)
    q = _sc_hop(h1, s1, d1, zeros_tile)

    bv3 = batch_vec.reshape(10, 1, 1000)
    eps_b = jnp.broadcast_to(eps.reshape(1, 1), (1, H))
    wpt = jnp.pad(Wp.T, ((0, 0), (0, H - DOUT)))
    bp_pad = jnp.pad(bp, (0, H - DOUT)).reshape(1, H)

    pred_pad = _head(bv3, h0, h1[:N], q[0, :N], q[1, :N],
                     eps_b, wpt, bp_pad)
    return pred_pad[:, :DOUT]


# speedup vs baseline: 2.9921x; 2.9921x over previous
import functools

import jax
import jax.numpy as jnp
from jax import lax
from jax.experimental import pallas as pl
from jax.experimental.pallas import tpu as pltpu
from jax.experimental.pallas import tpu_sc as plsc

N = 10000
E = 320000
H = 128
G = 128
DOUT = 10

NW = 32
LANES = 128
ROWS_PER_W = 80
E_PAD = NW * ROWS_PER_W * LANES
N_PAD = 10240
TILE_ROWS = N_PAD // 16
DUMP_ROW = N


def _hop_body(h_hbm, scat_hbm, dst_hbm, zeros_hbm, out_hbm,
              scat_v, dst_v, rows_v, acc_sh, sem):
    c = lax.axis_index("c")
    s = lax.axis_index("s")
    wid = s * 2 + c
    pltpu.sync_copy(zeros_hbm, acc_sh.at[pl.ds(s * TILE_ROWS, TILE_ROWS)])
    base = wid * ROWS_PER_W
    pltpu.sync_copy(scat_hbm.at[pl.ds(base, ROWS_PER_W)], scat_v)
    pltpu.sync_copy(dst_hbm.at[pl.ds(base, ROWS_PER_W)], dst_v)
    plsc.subcore_barrier()

    @pl.loop(0, ROWS_PER_W)
    def _(j):
        pltpu.async_copy(h_hbm.at[scat_v.at[j]], rows_v, sem).wait()
        pltpu.sync_copy(rows_v, acc_sh.at[dst_v.at[j]], add=True)

    plsc.subcore_barrier()
    pltpu.sync_copy(acc_sh.at[pl.ds(s * TILE_ROWS, TILE_ROWS)],
                    out_hbm.at[c, pl.ds(s * TILE_ROWS, TILE_ROWS)])


def _sc_hop(h, scat_rows, dst_rows, zeros_tile):
    mesh = plsc.VectorSubcoreMesh(core_axis_name="c", subcore_axis_name="s")
    return pl.kernel(
        _hop_body,
        out_type=jax.ShapeDtypeStruct((2, N_PAD, H), jnp.float32),
        mesh=mesh,
        scratch_types=[
            pltpu.VMEM((ROWS_PER_W, LANES), jnp.int32),
            pltpu.VMEM((ROWS_PER_W, LANES), jnp.int32),
            pltpu.VMEM((LANES, H), jnp.float32),
            pltpu.VMEM_SHARED((N_PAD, H), jnp.float32),
            pltpu.SemaphoreType.DMA,
        ],
    )(h, scat_rows, dst_rows, zeros_tile)


def _lin0_kernel(x_ref, w_ref, b_ref, o_ref):
    o_ref[...] = lax.dot_general(
        x_ref[...], w_ref[...], (((1,), (1,)), ((), ())),
        preferred_element_type=jnp.float32) + b_ref[...]


def _lin0(x, w0, b0):
    blk = 1000
    return pl.pallas_call(
        _lin0_kernel,
        grid=(N // blk,),
        in_specs=[
            pl.BlockSpec((blk, H), lambda i: (i, 0)),
            pl.BlockSpec((H, H), lambda i: (0, 0)),
            pl.BlockSpec((1, H), lambda i: (0, 0)),
        ],
        out_specs=pl.BlockSpec((blk, H), lambda i: (i, 0)),
        out_shape=jax.ShapeDtypeStruct((N, H), jnp.float32),
    )(x, w0, b0.reshape(1, H))


def _combine_kernel(a_ref, b_ref, o_ref):
    o_ref[...] = a_ref[...] + b_ref[...]


def _combine(p):
    blk = 1024
    return pl.pallas_call(
        _combine_kernel,
        grid=(N_PAD // blk,),
        in_specs=[
            pl.BlockSpec((blk, H), lambda i: (i, 0)),
            pl.BlockSpec((blk, H), lambda i: (i, 0)),
        ],
        out_specs=pl.BlockSpec((blk, H), lambda i: (i, 0)),
        out_shape=jax.ShapeDtypeStruct((N_PAD, H), jnp.float32),
    )(p[0], p[1])


def _head_kernel(bv_ref, h0_ref, h1_ref, q0_ref, q1_ref, eps_ref, wpt_ref,
                 bp_ref, o_ref, acc_ref, cnt_ref, *, nblocks):
    i = pl.program_id(0)

    @pl.when(i == 0)
    def _():
        acc_ref[...] = jnp.zeros_like(acc_ref)
        cnt_ref[...] = jnp.zeros_like(cnt_ref)

    bv = bv_ref[0]
    gids = lax.broadcasted_iota(jnp.int32, (G, bv.shape[1]), 0)
    s_mat = jnp.where(bv == gids, 1.0, 0.0)
    dot = functools.partial(lax.dot_general,
                            dimension_numbers=(((1,), (0,)), ((), ())),
                            preferred_element_type=jnp.float32)
    h2blk = q0_ref[...] + q1_ref[...]
    acc_ref[:, 0:H] += dot(s_mat, h0_ref[...])
    acc_ref[:, H:2 * H] += dot(s_mat, h1_ref[...])
    acc_ref[:, 2 * H:3 * H] += dot(s_mat, h2blk)
    cnt_ref[...] += jnp.broadcast_to(
        jnp.sum(s_mat, axis=1, keepdims=True), cnt_ref.shape)

    @pl.when(i == nblocks - 1)
    def _():
        c = jnp.maximum(cnt_ref[...], 1.0)
        scale = 1.0 + eps_ref[...]
        pooled = jnp.concatenate(
            [acc_ref[:, 0:H] * scale / c,
             acc_ref[:, H:2 * H] / c,
             acc_ref[:, 2 * H:3 * H] / c], axis=1)
        o_ref[...] = lax.dot_general(
            pooled, wpt_ref[...], (((1,), (0,)), ((), ())),
            preferred_element_type=jnp.float32) + bp_ref[...]


def _head(bv3, h0, h1, q0, q1, eps_b, wpt, bp_pad):
    blk = 1000
    nb = N // blk
    return pl.pallas_call(
        functools.partial(_head_kernel, nblocks=nb),
        grid=(nb,),
        in_specs=[
            pl.BlockSpec((1, 1, blk), lambda i: (i, 0, 0)),
            pl.BlockSpec((blk, H), lambda i: (i, 0)),
            pl.BlockSpec((blk, H), lambda i: (i, 0)),
            pl.BlockSpec((blk, H), lambda i: (i, 0)),
            pl.BlockSpec((blk, H), lambda i: (i, 0)),
            pl.BlockSpec((1, H), lambda i: (0, 0)),
            pl.BlockSpec((3 * H, H), lambda i: (0, 0)),
            pl.BlockSpec((1, H), lambda i: (0, 0)),
        ],
        out_specs=pl.BlockSpec((G, H), lambda i: (0, 0)),
        out_shape=jax.ShapeDtypeStruct((G, H), jnp.float32),
        scratch_shapes=[
            pltpu.VMEM((G, 3 * H), jnp.float32),
            pltpu.VMEM((G, H), jnp.float32),
        ],
    )(bv3, h0, h1, q0, q1, eps_b, wpt, bp_pad)


def _pad_edges(scat, dst):
    pad = E_PAD - E
    scat_p = jnp.concatenate(
        [scat, jnp.zeros((pad,), jnp.int32)]).reshape(E_PAD // LANES, LANES)
    dst_p = jnp.concatenate(
        [dst, jnp.full((pad,), DUMP_ROW, jnp.int32)]).reshape(
            E_PAD // LANES, LANES)
    return scat_p, dst_p


def kernel(x, agg_scatter_0, agg_node_index_0, agg_scatter_1,
           agg_node_index_1, batch_vec, eps, W0, b0, Wp, bp):
    h0 = _lin0(x, W0, b0)

    zeros_tile = jnp.zeros((TILE_ROWS, H), jnp.float32)
    s0, d0 = _pad_edges(agg_scatter_0, agg_node_index_0)
    s1, d1 = _pad_edges(agg_scatter_1, agg_node_index_1)

    p = _sc_hop(h0, s0, d0, zeros_tile)
    h1 = _combine(p)
    q = _sc_hop(h1, s1, d1, zeros_tile)

    bv3 = batch_vec.reshape(10, 1, 1000)
    eps_b = jnp.broadcast_to(eps.reshape(1, 1), (1, H))
    wpt = jnp.pad(Wp.T, ((0, 0), (0, H - DOUT)))
    bp_pad = jnp.pad(bp, (0, H - DOUT)).reshape(1, H)

    pred_pad = _head(bv3, h0, h1[:N], q[0, :N], q[1, :N],
                     eps_b, wpt, bp_pad)
    return pred_pad[:, :DOUT]


# speedup vs baseline: 3.3203x; 1.1097x over previous
import functools

import jax
import jax.numpy as jnp
from jax import lax
from jax.experimental import pallas as pl
from jax.experimental.pallas import tpu as pltpu
from jax.experimental.pallas import tpu_sc as plsc

N = 10000
E = 320000
H = 128
G = 128
DOUT = 10

NW = 32
LANES = 128
ROWS_PER_W = 80
E_PAD = NW * ROWS_PER_W * LANES
N_PAD = 10240
TILE_ROWS = N_PAD // 16
DUMP_ROW = N


GROUPS = 2
CPG = ROWS_PER_W // GROUPS


def _hop_body(h_hbm, scat_hbm, dst_hbm, zeros_hbm, out_hbm,
              scat_v, dst_v, rows_a, rows_b, acc_sh, sem_a, sem_b):
    rows = (rows_a, rows_b)
    sems = (sem_a, sem_b)
    c = lax.axis_index("c")
    s = lax.axis_index("s")
    wid = s * 2 + c
    pltpu.sync_copy(zeros_hbm, acc_sh.at[pl.ds(s * TILE_ROWS, TILE_ROWS)])
    plsc.subcore_barrier()

    @pl.loop(0, GROUPS)
    def _(g):
        base = wid * ROWS_PER_W + g * CPG
        pltpu.sync_copy(scat_hbm.at[pl.ds(base, CPG)], scat_v)
        pltpu.sync_copy(dst_hbm.at[pl.ds(base, CPG)], dst_v)

        def fire(j, b):
            pltpu.async_copy(h_hbm.at[scat_v.at[j]], rows[b], sems[b])

        fire(0, 0)
        fire(1, 1)

        @pl.loop(0, CPG // 2)
        def _(t):
            for b in range(2):
                j = t * 2 + b
                pltpu.make_async_copy(
                    h_hbm.at[scat_v.at[j]], rows[b], sems[b]).wait()
                pltpu.sync_copy(rows[b], acc_sh.at[dst_v.at[j]], add=True)

                @pl.when(j + 2 < CPG)
                def _():
                    fire(j + 2, b)

    plsc.subcore_barrier()
    pltpu.sync_copy(acc_sh.at[pl.ds(s * TILE_ROWS, TILE_ROWS)],
                    out_hbm.at[c, pl.ds(s * TILE_ROWS, TILE_ROWS)])


def _sc_hop(h, scat_rows, dst_rows, zeros_tile):
    mesh = plsc.VectorSubcoreMesh(core_axis_name="c", subcore_axis_name="s")
    return pl.kernel(
        _hop_body,
        out_type=jax.ShapeDtypeStruct((2, N_PAD, H), jnp.float32),
        mesh=mesh,
        scratch_types=[
            pltpu.VMEM((CPG, LANES), jnp.int32),
            pltpu.VMEM((CPG, LANES), jnp.int32),
            pltpu.VMEM((LANES, H), jnp.float32),
            pltpu.VMEM((LANES, H), jnp.float32),
            pltpu.VMEM_SHARED((N_PAD, H), jnp.float32),
            pltpu.SemaphoreType.DMA,
            pltpu.SemaphoreType.DMA,
        ],
    )(h, scat_rows, dst_rows, zeros_tile)


def _lin0_kernel(x_ref, w_ref, b_ref, o_ref):
    o_ref[...] = lax.dot_general(
        x_ref[...], w_ref[...], (((1,), (1,)), ((), ())),
        preferred_element_type=jnp.float32) + b_ref[...]


def _lin0(x, w0, b0):
    blk = 1000
    return pl.pallas_call(
        _lin0_kernel,
        grid=(N // blk,),
        in_specs=[
            pl.BlockSpec((blk, H), lambda i: (i, 0)),
            pl.BlockSpec((H, H), lambda i: (0, 0)),
            pl.BlockSpec((1, H), lambda i: (0, 0)),
        ],
        out_specs=pl.BlockSpec((blk, H), lambda i: (i, 0)),
        out_shape=jax.ShapeDtypeStruct((N, H), jnp.float32),
    )(x, w0, b0.reshape(1, H))


def _combine_kernel(a_ref, b_ref, o_ref):
    o_ref[...] = a_ref[...] + b_ref[...]


def _combine(p):
    blk = 1024
    return pl.pallas_call(
        _combine_kernel,
        grid=(N_PAD // blk,),
        in_specs=[
            pl.BlockSpec((blk, H), lambda i: (i, 0)),
            pl.BlockSpec((blk, H), lambda i: (i, 0)),
        ],
        out_specs=pl.BlockSpec((blk, H), lambda i: (i, 0)),
        out_shape=jax.ShapeDtypeStruct((N_PAD, H), jnp.float32),
    )(p[0], p[1])


def _head_kernel(bv_ref, h0_ref, h1_ref, q0_ref, q1_ref, eps_ref, wpt_ref,
                 bp_ref, o_ref, acc_ref, cnt_ref, *, nblocks):
    i = pl.program_id(0)

    @pl.when(i == 0)
    def _():
        acc_ref[...] = jnp.zeros_like(acc_ref)
        cnt_ref[...] = jnp.zeros_like(cnt_ref)

    bv = bv_ref[0]
    gids = lax.broadcasted_iota(jnp.int32, (G, bv.shape[1]), 0)
    s_mat = jnp.where(bv == gids, 1.0, 0.0)
    dot = functools.partial(lax.dot_general,
                            dimension_numbers=(((1,), (0,)), ((), ())),
                            preferred_element_type=jnp.float32)
    h2blk = q0_ref[...] + q1_ref[...]
    acc_ref[:, 0:H] += dot(s_mat, h0_ref[...])
    acc_ref[:, H:2 * H] += dot(s_mat, h1_ref[...])
    acc_ref[:, 2 * H:3 * H] += dot(s_mat, h2blk)
    cnt_ref[...] += jnp.broadcast_to(
        jnp.sum(s_mat, axis=1, keepdims=True), cnt_ref.shape)

    @pl.when(i == nblocks - 1)
    def _():
        c = jnp.maximum(cnt_ref[...], 1.0)
        scale = 1.0 + eps_ref[...]
        pooled = jnp.concatenate(
            [acc_ref[:, 0:H] * scale / c,
             acc_ref[:, H:2 * H] / c,
             acc_ref[:, 2 * H:3 * H] / c], axis=1)
        o_ref[...] = lax.dot_general(
            pooled, wpt_ref[...], (((1,), (0,)), ((), ())),
            preferred_element_type=jnp.float32) + bp_ref[...]


def _head(bv3, h0, h1, q0, q1, eps_b, wpt, bp_pad):
    blk = 1000
    nb = N // blk
    return pl.pallas_call(
        functools.partial(_head_kernel, nblocks=nb),
        grid=(nb,),
        in_specs=[
            pl.BlockSpec((1, 1, blk), lambda i: (i, 0, 0)),
            pl.BlockSpec((blk, H), lambda i: (i, 0)),
            pl.BlockSpec((blk, H), lambda i: (i, 0)),
            pl.BlockSpec((blk, H), lambda i: (i, 0)),
            pl.BlockSpec((blk, H), lambda i: (i, 0)),
            pl.BlockSpec((1, H), lambda i: (0, 0)),
            pl.BlockSpec((3 * H, H), lambda i: (0, 0)),
            pl.BlockSpec((1, H), lambda i: (0, 0)),
        ],
        out_specs=pl.BlockSpec((G, H), lambda i: (0, 0)),
        out_shape=jax.ShapeDtypeStruct((G, H), jnp.float32),
        scratch_shapes=[
            pltpu.VMEM((G, 3 * H), jnp.float32),
            pltpu.VMEM((G, H), jnp.float32),
        ],
    )(bv3, h0, h1, q0, q1, eps_b, wpt, bp_pad)


def _pad_edges(scat, dst):
    pad = E_PAD - E
    scat_p = jnp.concatenate(
        [scat, jnp.zeros((pad,), jnp.int32)]).reshape(E_PAD // LANES, LANES)
    dst_p = jnp.concatenate(
        [dst, jnp.full((pad,), DUMP_ROW, jnp.int32)]).reshape(
            E_PAD // LANES, LANES)
    return scat_p, dst_p


def kernel(x, agg_scatter_0, agg_node_index_0, agg_scatter_1,
           agg_node_index_1, batch_vec, eps, W0, b0, Wp, bp):
    h0 = _lin0(x, W0, b0)

    zeros_tile = jnp.zeros((TILE_ROWS, H), jnp.float32)
    s0, d0 = _pad_edges(agg_scatter_0, agg_node_index_0)
    s1, d1 = _pad_edges(agg_scatter_1, agg_node_index_1)

    p = _sc_hop(h0, s0, d0, zeros_tile)
    h1 = _combine(p)
    q = _sc_hop(h1, s1, d1, zeros_tile)

    bv3 = batch_vec.reshape(10, 1, 1000)
    eps_b = jnp.broadcast_to(eps.reshape(1, 1), (1, H))
    wpt = jnp.pad(Wp.T, ((0, 0), (0, H - DOUT)))
    bp_pad = jnp.pad(bp, (0, H - DOUT)).reshape(1, H)

    pred_pad = _head(bv3, h0, h1[:N], q[0, :N], q[1, :N],
                     eps_b, wpt, bp_pad)
    return pred_pad[:, :DOUT]


# speedup vs baseline: 12.0763x; 3.6371x over previous
import functools

import jax
import jax.numpy as jnp
from jax import lax
from jax.experimental import pallas as pl
from jax.experimental.pallas import tpu as pltpu
from jax.experimental.pallas import tpu_sc as plsc

N = 10000
E = 320000
H = 128
G = 128
DOUT = 10

NW = 32
LANES = 128
ROWS_PER_W = 80
E_PAD = NW * ROWS_PER_W * LANES
N_PAD = 10240
TILE_ROWS = N_PAD // 16
DUMP_ROW = N


GROUPS = 2
CPG = ROWS_PER_W // GROUPS


def _hop_body(h_hbm, scat_hbm, dst_hbm, zeros_hbm, out_hbm,
              scat_v, dst_v, rows_a, rows_b, acc_sh, sem_a, sem_b):
    rows = (rows_a, rows_b)
    sems = (sem_a, sem_b)
    c = lax.axis_index("c")
    s = lax.axis_index("s")
    wid = s * 2 + c
    pltpu.sync_copy(zeros_hbm, acc_sh.at[pl.ds(s * TILE_ROWS, TILE_ROWS)])
    plsc.subcore_barrier()

    @pl.loop(0, GROUPS)
    def _(g):
        base = wid * ROWS_PER_W + g * CPG
        pltpu.sync_copy(scat_hbm.at[pl.ds(base, CPG)], scat_v)
        pltpu.sync_copy(dst_hbm.at[pl.ds(base, CPG)], dst_v)

        def fire(j, b):
            pltpu.async_copy(h_hbm.at[scat_v.at[j]], rows[b], sems[b])

        fire(0, 0)
        fire(1, 1)

        @pl.loop(0, CPG // 2)
        def _(t):
            for b in range(2):
                j = t * 2 + b
                pltpu.make_async_copy(
                    h_hbm.at[scat_v.at[j]], rows[b], sems[b]).wait()
                pltpu.sync_copy(rows[b], acc_sh.at[dst_v.at[j]], add=True)

                @pl.when(j + 2 < CPG)
                def _():
                    fire(j + 2, b)

    plsc.subcore_barrier()
    pltpu.sync_copy(acc_sh.at[pl.ds(s * TILE_ROWS, TILE_ROWS)],
                    out_hbm.at[c, pl.ds(s * TILE_ROWS, TILE_ROWS)])


def _sc_hop(h, scat_rows, dst_rows, zeros_tile):
    mesh = plsc.VectorSubcoreMesh(core_axis_name="c", subcore_axis_name="s")
    return pl.kernel(
        _hop_body,
        out_type=jax.ShapeDtypeStruct((2, N_PAD, H), jnp.float32),
        mesh=mesh,
        scratch_types=[
            pltpu.VMEM((CPG, LANES), jnp.int32),
            pltpu.VMEM((CPG, LANES), jnp.int32),
            pltpu.VMEM((LANES, H), jnp.float32),
            pltpu.VMEM((LANES, H), jnp.float32),
            pltpu.VMEM_SHARED((N_PAD, H), jnp.float32),
            pltpu.SemaphoreType.DMA,
            pltpu.SemaphoreType.DMA,
        ],
    )(h, scat_rows, dst_rows, zeros_tile)


def _lin0_kernel(x_ref, w_ref, b_ref, o_ref):
    o_ref[...] = lax.dot_general(
        x_ref[...], w_ref[...], (((1,), (1,)), ((), ())),
        preferred_element_type=jnp.float32) + b_ref[...]


def _lin0(x, w0, b0):
    blk = 1000
    return pl.pallas_call(
        _lin0_kernel,
        grid=(N // blk,),
        in_specs=[
            pl.BlockSpec((blk, H), lambda i: (i, 0)),
            pl.BlockSpec((H, H), lambda i: (0, 0)),
            pl.BlockSpec((1, H), lambda i: (0, 0)),
        ],
        out_specs=pl.BlockSpec((blk, H), lambda i: (i, 0)),
        out_shape=jax.ShapeDtypeStruct((N, H), jnp.float32),
    )(x, w0, b0.reshape(1, H))


def _combine_kernel(a_ref, b_ref, o_ref):
    o_ref[...] = a_ref[...] + b_ref[...]


def _combine(p):
    blk = 1024
    return pl.pallas_call(
        _combine_kernel,
        grid=(N_PAD // blk,),
        in_specs=[
            pl.BlockSpec((blk, H), lambda i: (i, 0)),
            pl.BlockSpec((blk, H), lambda i: (i, 0)),
        ],
        out_specs=pl.BlockSpec((blk, H), lambda i: (i, 0)),
        out_shape=jax.ShapeDtypeStruct((N_PAD, H), jnp.float32),
    )(p[0], p[1])


def _head_kernel(bv_ref, h0_ref, h1_ref, q0_ref, q1_ref, eps_ref, wpt_ref,
                 bp_ref, o_ref, acc_ref, cnt_ref, *, nblocks):
    i = pl.program_id(0)

    @pl.when(i == 0)
    def _():
        acc_ref[...] = jnp.zeros_like(acc_ref)
        cnt_ref[...] = jnp.zeros_like(cnt_ref)

    bv = bv_ref[0]
    gids = lax.broadcasted_iota(jnp.int32, (G, bv.shape[1]), 0)
    s_mat = jnp.where(bv == gids, 1.0, 0.0)
    dot = functools.partial(lax.dot_general,
                            dimension_numbers=(((1,), (0,)), ((), ())),
                            preferred_element_type=jnp.float32)
    h2blk = q0_ref[...] + q1_ref[...]
    acc_ref[:, 0:H] += dot(s_mat, h0_ref[...])
    acc_ref[:, H:2 * H] += dot(s_mat, h1_ref[...])
    acc_ref[:, 2 * H:3 * H] += dot(s_mat, h2blk)
    cnt_ref[...] += jnp.broadcast_to(
        jnp.sum(s_mat, axis=1, keepdims=True), cnt_ref.shape)

    @pl.when(i == nblocks - 1)
    def _():
        c = jnp.maximum(cnt_ref[...], 1.0)
        scale = 1.0 + eps_ref[...]
        pooled = jnp.concatenate(
            [acc_ref[:, 0:H] * scale / c,
             acc_ref[:, H:2 * H] / c,
             acc_ref[:, 2 * H:3 * H] / c], axis=1)
        o_ref[...] = lax.dot_general(
            pooled, wpt_ref[...], (((1,), (0,)), ((), ())),
            preferred_element_type=jnp.float32) + bp_ref[...]


def _head(bv3, h0, h1, q0, q1, eps_b, wpt, bp_pad):
    blk = 1000
    nb = N // blk
    return pl.pallas_call(
        functools.partial(_head_kernel, nblocks=nb),
        grid=(nb,),
        in_specs=[
            pl.BlockSpec((1, 1, blk), lambda i: (i, 0, 0)),
            pl.BlockSpec((blk, H), lambda i: (i, 0)),
            pl.BlockSpec((blk, H), lambda i: (i, 0)),
            pl.BlockSpec((blk, H), lambda i: (i, 0)),
            pl.BlockSpec((blk, H), lambda i: (i, 0)),
            pl.BlockSpec((1, H), lambda i: (0, 0)),
            pl.BlockSpec((3 * H, H), lambda i: (0, 0)),
            pl.BlockSpec((1, H), lambda i: (0, 0)),
        ],
        out_specs=pl.BlockSpec((G, H), lambda i: (0, 0)),
        out_shape=jax.ShapeDtypeStruct((G, H), jnp.float32),
        scratch_shapes=[
            pltpu.VMEM((G, 3 * H), jnp.float32),
            pltpu.VMEM((G, H), jnp.float32),
        ],
    )(bv3, h0, h1, q0, q1, eps_b, wpt, bp_pad)


def _pad_edges(scat, dst):
    pad = E_PAD - E
    r = jnp.arange(pad, dtype=jnp.int32)
    scat_p = jnp.concatenate(
        [scat, r % N]).reshape(E_PAD // LANES, LANES)
    dst_p = jnp.concatenate(
        [dst, DUMP_ROW + r % (N_PAD - N)]).reshape(E_PAD // LANES, LANES)
    return scat_p, dst_p


def kernel(x, agg_scatter_0, agg_node_index_0, agg_scatter_1,
           agg_node_index_1, batch_vec, eps, W0, b0, Wp, bp):
    h0 = _lin0(x, W0, b0)

    zeros_tile = jnp.zeros((TILE_ROWS, H), jnp.float32)
    s0, d0 = _pad_edges(agg_scatter_0, agg_node_index_0)
    s1, d1 = _pad_edges(agg_scatter_1, agg_node_index_1)

    p = _sc_hop(h0, s0, d0, zeros_tile)
    h1 = _combine(p)
    q = _sc_hop(h1, s1, d1, zeros_tile)

    bv3 = batch_vec.reshape(10, 1, 1000)
    eps_b = jnp.broadcast_to(eps.reshape(1, 1), (1, H))
    wpt = jnp.pad(Wp.T, ((0, 0), (0, H - DOUT)))
    bp_pad = jnp.pad(bp, (0, H - DOUT)).reshape(1, H)

    pred_pad = _head(bv3, h0, h1[:N], q[0, :N], q[1, :N],
                     eps_b, wpt, bp_pad)
    return pred_pad[:, :DOUT]


# speedup vs baseline: 13.0052x; 1.0769x over previous
import functools

import jax
import jax.numpy as jnp
from jax import lax
from jax.experimental import pallas as pl
from jax.experimental.pallas import tpu as pltpu
from jax.experimental.pallas import tpu_sc as plsc

N = 10000
E = 320000
H = 128
G = 128
DOUT = 10

NW = 32
LANES = 128
ROWS_PER_W = 80
E_PAD = NW * ROWS_PER_W * LANES
N_PAD = 10240
TILE_ROWS = N_PAD // 16
DUMP_ROW = N
BLK = 1000


GROUPS = 2
CPG = ROWS_PER_W // GROUPS
ZROWS = 16


def _hop_body(h_hbm, scat_hbm, dst_hbm, zeros_hbm, out_hbm,
              scat_v, dst_v, rows_a, rows_b, zbuf, acc_sh, sem_a, sem_b):
    rows = (rows_a, rows_b)
    sems = (sem_a, sem_b)
    c = lax.axis_index("c")
    s = lax.axis_index("s")
    wid = s * 2 + c
    pltpu.sync_copy(zeros_hbm, zbuf)

    @pl.loop(0, TILE_ROWS // ZROWS)
    def _(k):
        pltpu.sync_copy(zbuf, acc_sh.at[pl.ds(s * TILE_ROWS + k * ZROWS,
                                              ZROWS)])

    plsc.subcore_barrier()

    @pl.loop(0, GROUPS)
    def _(g):
        base = wid * ROWS_PER_W + g * CPG
        pltpu.sync_copy(scat_hbm.at[pl.ds(base, CPG)], scat_v)
        pltpu.sync_copy(dst_hbm.at[pl.ds(base, CPG)], dst_v)

        def fire(j, b):
            pltpu.async_copy(h_hbm.at[scat_v.at[j]], rows[b], sems[b])

        fire(0, 0)
        fire(1, 1)

        @pl.loop(0, CPG // 2)
        def _(t):
            for b in range(2):
                j = t * 2 + b
                pltpu.make_async_copy(
                    h_hbm.at[scat_v.at[j]], rows[b], sems[b]).wait()
                pltpu.sync_copy(rows[b], acc_sh.at[dst_v.at[j]], add=True)

                @pl.when(j + 2 < CPG)
                def _():
                    fire(j + 2, b)

    plsc.subcore_barrier()
    pltpu.sync_copy(acc_sh.at[pl.ds(s * TILE_ROWS, TILE_ROWS)],
                    out_hbm.at[c, pl.ds(s * TILE_ROWS, TILE_ROWS)])


def _sc_hop(h, scat_rows, dst_rows, zeros_tile):
    mesh = plsc.VectorSubcoreMesh(core_axis_name="c", subcore_axis_name="s")
    return pl.kernel(
        _hop_body,
        out_type=jax.ShapeDtypeStruct((2, N_PAD, H), jnp.float32),
        mesh=mesh,
        scratch_types=[
            pltpu.VMEM((CPG, LANES), jnp.int32),
            pltpu.VMEM((CPG, LANES), jnp.int32),
            pltpu.VMEM((LANES, H), jnp.float32),
            pltpu.VMEM((LANES, H), jnp.float32),
            pltpu.VMEM((ZROWS, H), jnp.float32),
            pltpu.VMEM_SHARED((N_PAD, H), jnp.float32),
            pltpu.SemaphoreType.DMA,
            pltpu.SemaphoreType.DMA,
        ],
    )(h, scat_rows, dst_rows, zeros_tile)


_DOT = functools.partial(lax.dot_general,
                         dimension_numbers=(((1,), (0,)), ((), ())),
                         preferred_element_type=jnp.float32)


def _onehot(bv_ref):
    bv = bv_ref[0]
    gids = lax.broadcasted_iota(jnp.int32, (G, bv.shape[1]), 0)
    return jnp.where(bv == gids, 1.0, 0.0)


def _lin0_kernel(bv_ref, x_ref, w_ref, b_ref, o_ref, acc0_ref, cnt_ref,
                 accs, cnts, *, nblocks):
    i = pl.program_id(0)

    @pl.when(i == 0)
    def _():
        accs[...] = jnp.zeros_like(accs)
        cnts[...] = jnp.zeros_like(cnts)

    h = lax.dot_general(x_ref[...], w_ref[...], (((1,), (1,)), ((), ())),
                        preferred_element_type=jnp.float32) + b_ref[...]
    o_ref[...] = h
    s_mat = _onehot(bv_ref)
    accs[...] += _DOT(s_mat, h)
    cnts[...] += jnp.broadcast_to(
        jnp.sum(s_mat, axis=1, keepdims=True), cnts.shape)

    @pl.when(i == nblocks - 1)
    def _():
        acc0_ref[...] = accs[...]
        cnt_ref[...] = cnts[...]


def _lin0(bv3, x, w0, b0):
    nb = N // BLK
    return pl.pallas_call(
        functools.partial(_lin0_kernel, nblocks=nb),
        grid=(nb,),
        in_specs=[
            pl.BlockSpec((1, 1, BLK), lambda i: (i, 0, 0)),
            pl.BlockSpec((BLK, H), lambda i: (i, 0)),
            pl.BlockSpec((H, H), lambda i: (0, 0)),
            pl.BlockSpec((1, H), lambda i: (0, 0)),
        ],
        out_specs=[
            pl.BlockSpec((BLK, H), lambda i: (i, 0)),
            pl.BlockSpec((G, H), lambda i: (0, 0)),
            pl.BlockSpec((G, H), lambda i: (0, 0)),
        ],
        out_shape=[
            jax.ShapeDtypeStruct((N, H), jnp.float32),
            jax.ShapeDtypeStruct((G, H), jnp.float32),
            jax.ShapeDtypeStruct((G, H), jnp.float32),
        ],
        scratch_shapes=[
            pltpu.VMEM((G, H), jnp.float32),
            pltpu.VMEM((G, H), jnp.float32),
        ],
    )(bv3, x, w0, b0.reshape(1, H))


def _combine_kernel(bv_ref, a_ref, b_ref, o_ref, acc1_ref, accs, *, nblocks):
    i = pl.program_id(0)

    @pl.when(i == 0)
    def _():
        accs[...] = jnp.zeros_like(accs)

    hb = a_ref[0] + b_ref[0]
    o_ref[...] = hb
    accs[...] += _DOT(_onehot(bv_ref), hb)

    @pl.when(i == nblocks - 1)
    def _():
        acc1_ref[...] = accs[...]


def _combine(bv3, p):
    nb = N // BLK
    return pl.pallas_call(
        functools.partial(_combine_kernel, nblocks=nb),
        grid=(nb,),
        in_specs=[
            pl.BlockSpec((1, 1, BLK), lambda i: (i, 0, 0)),
            pl.BlockSpec((1, BLK, H), lambda i: (0, i, 0)),
            pl.BlockSpec((1, BLK, H), lambda i: (1, i, 0)),
        ],
        out_specs=[
            pl.BlockSpec((BLK, H), lambda i: (i, 0)),
            pl.BlockSpec((G, H), lambda i: (0, 0)),
        ],
        out_shape=[
            jax.ShapeDtypeStruct((N, H), jnp.float32),
            jax.ShapeDtypeStruct((G, H), jnp.float32),
        ],
        scratch_shapes=[pltpu.VMEM((G, H), jnp.float32)],
    )(bv3, p, p)


def _head_kernel(bv_ref, q0_ref, q1_ref, acc0_ref, acc1_ref, cnt_ref,
                 eps_ref, wpt_ref, bp_ref, o_ref, accs, *, nblocks):
    i = pl.program_id(0)

    @pl.when(i == 0)
    def _():
        accs[...] = jnp.zeros_like(accs)

    accs[...] += _DOT(_onehot(bv_ref), q0_ref[0] + q1_ref[0])

    @pl.when(i == nblocks - 1)
    def _():
        c = jnp.maximum(cnt_ref[...], 1.0)
        scale = 1.0 + eps_ref[...]
        pooled = jnp.concatenate(
            [acc0_ref[...] * scale / c,
             acc1_ref[...] / c,
             accs[...] / c], axis=1)
        o_ref[...] = lax.dot_general(
            pooled, wpt_ref[...], (((1,), (0,)), ((), ())),
            preferred_element_type=jnp.float32) + bp_ref[...]


def _head(bv3, q, acc0, acc1, cnt, eps_b, wpt, bp_pad):
    nb = N // BLK
    return pl.pallas_call(
        functools.partial(_head_kernel, nblocks=nb),
        grid=(nb,),
        in_specs=[
            pl.BlockSpec((1, 1, BLK), lambda i: (i, 0, 0)),
            pl.BlockSpec((1, BLK, H), lambda i: (0, i, 0)),
            pl.BlockSpec((1, BLK, H), lambda i: (1, i, 0)),
            pl.BlockSpec((G, H), lambda i: (0, 0)),
            pl.BlockSpec((G, H), lambda i: (0, 0)),
            pl.BlockSpec((G, H), lambda i: (0, 0)),
            pl.BlockSpec((1, H), lambda i: (0, 0)),
            pl.BlockSpec((3 * H, H), lambda i: (0, 0)),
            pl.BlockSpec((1, H), lambda i: (0, 0)),
        ],
        out_specs=pl.BlockSpec((G, H), lambda i: (0, 0)),
        out_shape=jax.ShapeDtypeStruct((G, H), jnp.float32),
        scratch_shapes=[pltpu.VMEM((G, H), jnp.float32)],
    )(bv3, q, q, acc0, acc1, cnt, eps_b, wpt, bp_pad)


def _pad_edges(scat, dst):
    pad = E_PAD - E
    r = jnp.arange(pad, dtype=jnp.int32)
    scat_p = jnp.concatenate(
        [scat, r % N]).reshape(E_PAD // LANES, LANES)
    dst_p = jnp.concatenate(
        [dst, DUMP_ROW + r % (N_PAD - N)]).reshape(E_PAD // LANES, LANES)
    return scat_p, dst_p


def kernel(x, agg_scatter_0, agg_node_index_0, agg_scatter_1,
           agg_node_index_1, batch_vec, eps, W0, b0, Wp, bp):
    bv3 = batch_vec.reshape(N // BLK, 1, BLK)
    h0, acc0, cnt = _lin0(bv3, x, W0, b0)

    zeros_tile = jnp.zeros((ZROWS, H), jnp.float32)
    s0, d0 = _pad_edges(agg_scatter_0, agg_node_index_0)
    s1, d1 = _pad_edges(agg_scatter_1, agg_node_index_1)

    p = _sc_hop(h0, s0, d0, zeros_tile)
    h1, acc1 = _combine(bv3, p)
    q = _sc_hop(h1, s1, d1, zeros_tile)

    eps_b = jnp.broadcast_to(eps.reshape(1, 1), (1, H))
    wpt = jnp.pad(Wp.T, ((0, 0), (0, H - DOUT)))
    bp_pad = jnp.pad(bp, (0, H - DOUT)).reshape(1, H)

    pred_pad = _head(bv3, q, acc0, acc1, cnt, eps_b, wpt, bp_pad)
    return pred_pad[:, :DOUT]


# speedup vs baseline: 13.1226x; 1.0090x over previous
import functools

import jax
import jax.numpy as jnp
from jax import lax
from jax.experimental import pallas as pl
from jax.experimental.pallas import tpu as pltpu
from jax.experimental.pallas import tpu_sc as plsc

N = 10000
E = 320000
H = 128
G = 128
DOUT = 10

NW = 32
LANES = 128
ROWS_PER_W = 80
E_PAD = NW * ROWS_PER_W * LANES
N_PAD = 10240
TILE_ROWS = N_PAD // 16
DUMP_ROW = N
BLK = 1000


GROUPS = 2
CPG = ROWS_PER_W // GROUPS
ZROWS = 16


def _hop_body(h_hbm, scat_hbm, dst_hbm, zeros_hbm, out_hbm,
              scat_v, dst_v, rows_a, rows_b, zbuf, acc_sh, sem_a, sem_b):
    rows = (rows_a, rows_b)
    sems = (sem_a, sem_b)
    c = lax.axis_index("c")
    s = lax.axis_index("s")
    wid = s * 2 + c
    def stage(g):
        base = wid * ROWS_PER_W + g * CPG
        pltpu.sync_copy(scat_hbm.at[pl.ds(base, CPG)], scat_v)
        pltpu.sync_copy(dst_hbm.at[pl.ds(base, CPG)], dst_v)

    def fire(j, b):
        pltpu.async_copy(h_hbm.at[scat_v.at[j]], rows[b], sems[b])

    stage(0)
    fire(0, 0)
    fire(1, 1)

    pltpu.sync_copy(zeros_hbm, zbuf)

    @pl.loop(0, TILE_ROWS // ZROWS)
    def _(k):
        pltpu.sync_copy(zbuf, acc_sh.at[pl.ds(s * TILE_ROWS + k * ZROWS,
                                              ZROWS)])

    plsc.subcore_barrier()

    for g in range(GROUPS):
        if g:
            stage(g)
            fire(0, 0)
            fire(1, 1)

        @pl.loop(0, CPG // 2)
        def _(t):
            for b in range(2):
                j = t * 2 + b
                pltpu.make_async_copy(
                    h_hbm.at[scat_v.at[j]], rows[b], sems[b]).wait()
                pltpu.sync_copy(rows[b], acc_sh.at[dst_v.at[j]], add=True)

                @pl.when(j + 2 < CPG)
                def _():
                    fire(j + 2, b)

    plsc.subcore_barrier()
    pltpu.sync_copy(acc_sh.at[pl.ds(s * TILE_ROWS, TILE_ROWS)],
                    out_hbm.at[c, pl.ds(s * TILE_ROWS, TILE_ROWS)])


def _sc_hop(h, scat_rows, dst_rows, zeros_tile):
    mesh = plsc.VectorSubcoreMesh(core_axis_name="c", subcore_axis_name="s")
    return pl.kernel(
        _hop_body,
        out_type=jax.ShapeDtypeStruct((2, N_PAD, H), jnp.float32),
        mesh=mesh,
        scratch_types=[
            pltpu.VMEM((CPG, LANES), jnp.int32),
            pltpu.VMEM((CPG, LANES), jnp.int32),
            pltpu.VMEM((LANES, H), jnp.float32),
            pltpu.VMEM((LANES, H), jnp.float32),
            pltpu.VMEM((ZROWS, H), jnp.float32),
            pltpu.VMEM_SHARED((N_PAD, H), jnp.float32),
            pltpu.SemaphoreType.DMA,
            pltpu.SemaphoreType.DMA,
        ],
    )(h, scat_rows, dst_rows, zeros_tile)


_DOT = functools.partial(lax.dot_general,
                         dimension_numbers=(((1,), (0,)), ((), ())),
                         preferred_element_type=jnp.float32)


def _onehot(bv_ref):
    bv = bv_ref[0]
    gids = lax.broadcasted_iota(jnp.int32, (G, bv.shape[1]), 0)
    return jnp.where(bv == gids, 1.0, 0.0)


def _lin0_kernel(bv_ref, x_ref, w_ref, b_ref, o_ref, acc0_ref, cnt_ref,
                 accs, cnts, *, nblocks):
    i = pl.program_id(0)

    @pl.when(i == 0)
    def _():
        accs[...] = jnp.zeros_like(accs)
        cnts[...] = jnp.zeros_like(cnts)

    h = lax.dot_general(x_ref[...], w_ref[...], (((1,), (1,)), ((), ())),
                        preferred_element_type=jnp.float32) + b_ref[...]
    o_ref[...] = h
    s_mat = _onehot(bv_ref)
    accs[...] += _DOT(s_mat, h)
    cnts[...] += jnp.broadcast_to(
        jnp.sum(s_mat, axis=1, keepdims=True), cnts.shape)

    @pl.when(i == nblocks - 1)
    def _():
        acc0_ref[...] = accs[...]
        cnt_ref[...] = cnts[...]


def _lin0(bv3, x, w0, b0):
    nb = N // BLK
    return pl.pallas_call(
        functools.partial(_lin0_kernel, nblocks=nb),
        grid=(nb,),
        in_specs=[
            pl.BlockSpec((1, 1, BLK), lambda i: (i, 0, 0)),
            pl.BlockSpec((BLK, H), lambda i: (i, 0)),
            pl.BlockSpec((H, H), lambda i: (0, 0)),
            pl.BlockSpec((1, H), lambda i: (0, 0)),
        ],
        out_specs=[
            pl.BlockSpec((BLK, H), lambda i: (i, 0)),
            pl.BlockSpec((G, H), lambda i: (0, 0)),
            pl.BlockSpec((G, H), lambda i: (0, 0)),
        ],
        out_shape=[
            jax.ShapeDtypeStruct((N, H), jnp.float32),
            jax.ShapeDtypeStruct((G, H), jnp.float32),
            jax.ShapeDtypeStruct((G, H), jnp.float32),
        ],
        scratch_shapes=[
            pltpu.VMEM((G, H), jnp.float32),
            pltpu.VMEM((G, H), jnp.float32),
        ],
    )(bv3, x, w0, b0.reshape(1, H))


def _combine_kernel(bv_ref, a_ref, b_ref, o_ref, acc1_ref, accs, *, nblocks):
    i = pl.program_id(0)

    @pl.when(i == 0)
    def _():
        accs[...] = jnp.zeros_like(accs)

    hb = a_ref[0] + b_ref[0]
    o_ref[...] = hb
    accs[...] += _DOT(_onehot(bv_ref), hb)

    @pl.when(i == nblocks - 1)
    def _():
        acc1_ref[...] = accs[...]


def _combine(bv3, p):
    nb = N // BLK
    return pl.pallas_call(
        functools.partial(_combine_kernel, nblocks=nb),
        grid=(nb,),
        in_specs=[
            pl.BlockSpec((1, 1, BLK), lambda i: (i, 0, 0)),
            pl.BlockSpec((1, BLK, H), lambda i: (0, i, 0)),
            pl.BlockSpec((1, BLK, H), lambda i: (1, i, 0)),
        ],
        out_specs=[
            pl.BlockSpec((BLK, H), lambda i: (i, 0)),
            pl.BlockSpec((G, H), lambda i: (0, 0)),
        ],
        out_shape=[
            jax.ShapeDtypeStruct((N, H), jnp.float32),
            jax.ShapeDtypeStruct((G, H), jnp.float32),
        ],
        scratch_shapes=[pltpu.VMEM((G, H), jnp.float32)],
    )(bv3, p, p)


def _head_kernel(bv_ref, q0_ref, q1_ref, acc0_ref, acc1_ref, cnt_ref,
                 eps_ref, wpt_ref, bp_ref, o_ref, accs, *, nblocks):
    i = pl.program_id(0)

    @pl.when(i == 0)
    def _():
        accs[...] = jnp.zeros_like(accs)

    accs[...] += _DOT(_onehot(bv_ref), q0_ref[0] + q1_ref[0])

    @pl.when(i == nblocks - 1)
    def _():
        c = jnp.maximum(cnt_ref[...], 1.0)
        scale = 1.0 + eps_ref[...]
        pooled = jnp.concatenate(
            [acc0_ref[...] * scale / c,
             acc1_ref[...] / c,
             accs[...] / c], axis=1)
        o_ref[...] = lax.dot_general(
            pooled, wpt_ref[...], (((1,), (0,)), ((), ())),
            preferred_element_type=jnp.float32) + bp_ref[...]


def _head(bv3, q, acc0, acc1, cnt, eps_b, wpt, bp_pad):
    nb = N // BLK
    return pl.pallas_call(
        functools.partial(_head_kernel, nblocks=nb),
        grid=(nb,),
        in_specs=[
            pl.BlockSpec((1, 1, BLK), lambda i: (i, 0, 0)),
            pl.BlockSpec((1, BLK, H), lambda i: (0, i, 0)),
            pl.BlockSpec((1, BLK, H), lambda i: (1, i, 0)),
            pl.BlockSpec((G, H), lambda i: (0, 0)),
            pl.BlockSpec((G, H), lambda i: (0, 0)),
            pl.BlockSpec((G, H), lambda i: (0, 0)),
            pl.BlockSpec((1, H), lambda i: (0, 0)),
            pl.BlockSpec((3 * H, H), lambda i: (0, 0)),
            pl.BlockSpec((1, H), lambda i: (0, 0)),
        ],
        out_specs=pl.BlockSpec((G, H), lambda i: (0, 0)),
        out_shape=jax.ShapeDtypeStruct((G, H), jnp.float32),
        scratch_shapes=[pltpu.VMEM((G, H), jnp.float32)],
    )(bv3, q, q, acc0, acc1, cnt, eps_b, wpt, bp_pad)


def _pad_edges(scat, dst):
    pad = E_PAD - E
    r = jnp.arange(pad, dtype=jnp.int32)
    scat_p = jnp.concatenate(
        [scat, r % N]).reshape(E_PAD // LANES, LANES)
    dst_p = jnp.concatenate(
        [dst, DUMP_ROW + r % (N_PAD - N)]).reshape(E_PAD // LANES, LANES)
    return scat_p, dst_p


def kernel(x, agg_scatter_0, agg_node_index_0, agg_scatter_1,
           agg_node_index_1, batch_vec, eps, W0, b0, Wp, bp):
    bv3 = batch_vec.reshape(N // BLK, 1, BLK)
    h0, acc0, cnt = _lin0(bv3, x, W0, b0)

    zeros_tile = jnp.zeros((ZROWS, H), jnp.float32)
    s0, d0 = _pad_edges(agg_scatter_0, agg_node_index_0)
    s1, d1 = _pad_edges(agg_scatter_1, agg_node_index_1)

    p = _sc_hop(h0, s0, d0, zeros_tile)
    h1, acc1 = _combine(bv3, p)
    q = _sc_hop(h1, s1, d1, zeros_tile)

    eps_b = jnp.broadcast_to(eps.reshape(1, 1), (1, H))
    wpt = jnp.pad(Wp.T, ((0, 0), (0, H - DOUT)))
    bp_pad = jnp.pad(bp, (0, H - DOUT)).reshape(1, H)

    pred_pad = _head(bv3, q, acc0, acc1, cnt, eps_b, wpt, bp_pad)
    return pred_pad[:, :DOUT]
